# baseline (speedup 1.0000x reference)
import functools

import numpy as np
import jax
import jax.numpy as jnp
from jax import lax
from jax.experimental import pallas as pl
from jax.experimental.pallas import tpu as pltpu
from jax.experimental.pallas import tpu_sc as plsc


def _gen_base_anchors():
    base_size = 16
    ratios = np.array([0.5, 1.0, 2.0], dtype=np.float64)
    scales = np.array([8.0, 16.0, 32.0], dtype=np.float64)
    base = np.array([0.0, 0.0, base_size - 1.0, base_size - 1.0])
    w = base[2] - base[0] + 1.0
    h = base[3] - base[1] + 1.0
    x_ctr = base[0] + 0.5 * (w - 1.0)
    y_ctr = base[1] + 0.5 * (h - 1.0)
    size = w * h
    ws_r = np.round(np.sqrt(size / ratios))
    hs_r = np.round(ws_r * ratios)
    anchors = []
    for i in range(3):
        w_i, h_i = ws_r[i], hs_r[i]
        for s in scales:
            ws, hs = w_i * s, h_i * s
            anchors.append([x_ctr - 0.5 * (ws - 1.0), y_ctr - 0.5 * (hs - 1.0),
                            x_ctr + 0.5 * (ws - 1.0), y_ctr + 0.5 * (hs - 1.0)])
    return np.array(anchors, dtype=np.float32)


_ANCH = _gen_base_anchors()
_AW = [float(a[2] - a[0] + 1.0) for a in _ANCH]
_AH = [float(a[3] - a[1] + 1.0) for a in _ANCH]
_ACX = [float(a[0] + 0.5 * (a[2] - a[0] + 1.0)) for a in _ANCH]
_ACY = [float(a[1] + 0.5 * (a[3] - a[1] + 1.0)) for a in _ANCH]

FEAT_STRIDE = 16
B, A, H, W = 16, 9, 64, 64
K = H * W
C4 = 4 * A
NC, NS, L = 2, 16, 16
NW = NC * NS
CELLS_PER_TILE = B * K // NW
CHUNK = 1024
GROUPS = CHUNK // L

_mesh = plsc.VectorSubcoreMesh(
    core_axis_name="c", subcore_axis_name="s", num_cores=NC, num_subcores=NS)


@functools.partial(
    pl.kernel,
    out_type=(jax.ShapeDtypeStruct((B * K * C4,), jnp.float32),
              jax.ShapeDtypeStruct((B * K * A,), jnp.float32)),
    mesh=_mesh,
    scratch_types=[
        pltpu.VMEM((C4, CHUNK), jnp.float32),
        pltpu.VMEM((A * CHUNK,), jnp.float32),
        pltpu.VMEM((CHUNK * C4,), jnp.float32),
        pltpu.VMEM((CHUNK * A,), jnp.float32),
        pltpu.SemaphoreType.DMA,
    ],
    compiler_params=pltpu.CompilerParams(needs_layout_passes=False),
)
def _proposal_sc(bbox_hbm, cls_hbm, props_hbm, scores_hbm, bb, cl, po, so, sem):
    wid = lax.axis_index("s") * NC + lax.axis_index("c")
    batch = wid // 2
    cell0 = (wid % 2) * CELLS_PER_TILE

    iota = lax.iota(jnp.int32, L)
    sx_lane = (iota * FEAT_STRIDE).astype(jnp.float32)
    idx36 = iota * C4
    idx9 = iota * A

    for chunk in range(CELLS_PER_TILE // CHUNK):
        base = cell0 + chunk * CHUNK
        score_copies = [
            pltpu.async_copy(
                cls_hbm.at[pl.ds((batch * 2 * A + A + a) * K + base, CHUNK)],
                cl.at[pl.ds(a * CHUNK, CHUNK)], sem)
            for a in range(A)
        ]
        pltpu.sync_copy(bbox_hbm.at[batch, :, pl.ds(base, CHUNK)], bb)
        for c in score_copies:
            c.wait()

        def group_body(g, carry, base=base):
            col = g * L
            gcell = base + col
            sx0 = ((gcell % W) * FEAT_STRIDE).astype(jnp.float32)
            sy0 = ((gcell // W) * FEAT_STRIDE).astype(jnp.float32)
            sx = sx_lane + sx0
            for a in range(A):
                dx = bb[4 * a + 0, pl.ds(col, L)]
                dy = bb[4 * a + 1, pl.ds(col, L)]
                dw = bb[4 * a + 2, pl.ds(col, L)]
                dh = bb[4 * a + 3, pl.ds(col, L)]
                px = dx * _AW[a] + (sx + _ACX[a])
                py = dy * _AH[a] + (sy0 + _ACY[a])
                hw = (jnp.exp(dw) * _AW[a]) * 0.5
                hh = (jnp.exp(dh) * _AH[a]) * 0.5
                idx = idx36 + (col * C4 + 4 * a)
                plsc.store_scatter(po, [idx], px - hw)
                plsc.store_scatter(po, [idx + 1], py - hh)
                plsc.store_scatter(po, [idx + 2], px + hw)
                plsc.store_scatter(po, [idx + 3], py + hh)
                plsc.store_scatter(so, [idx9 + (col * A + a)],
                                   cl[pl.ds(a * CHUNK + col, L)])
            return carry

        lax.fori_loop(0, GROUPS, group_body, 0)

        pltpu.sync_copy(
            po, props_hbm.at[pl.ds((batch * K + base) * C4, CHUNK * C4)])
        pltpu.sync_copy(
            so, scores_hbm.at[pl.ds((batch * K + base) * A, CHUNK * A)])


def kernel(rpn_cls_probs, rpn_pred_bboxes, im_shapes, cfg_key):
    del im_shapes, cfg_key
    bbox = rpn_pred_bboxes.reshape(B, C4, K)
    cls = rpn_cls_probs.reshape(B * 2 * A * K)
    props, scores = _proposal_sc(bbox, cls)
    return props.reshape(B, K * A, 4), scores.reshape(B, K * A)
